# Initial kernel scaffold; baseline (speedup 1.0000x reference)
import functools

import jax
import jax.numpy as jnp
from jax import lax
from jax.experimental import pallas as pl
from jax.experimental.pallas import tpu as pltpu
from jax.experimental.pallas import tpu_sc as plsc

N_NODES = 10000
N_EDGES = 320000
D_IN = 128
D_OUT = 128
D_PAD = 144

NUM_CORES = 2
NUM_SUBCORES = 16
NUM_TILES = NUM_CORES * NUM_SUBCORES

E_PER_TILE = N_EDGES // NUM_TILES
CHUNK = 80
N_CHUNKS = E_PER_TILE // CHUNK
ROWS_PER_TILE = N_NODES // NUM_SUBCORES
STAGE_ROWS = 125


def _sc_body(xa_hbm, src_hbm, dst_hbm, zeros_hbm, parts_hbm,
             src_v, dst_v, rows_v, stage_v, acc_sh, sem):
    c = lax.axis_index("c")
    s = lax.axis_index("s")
    wid = c * NUM_SUBCORES + s
    ebase = wid * E_PER_TILE
    rbase = s * ROWS_PER_TILE

    pltpu.sync_copy(zeros_hbm, stage_v)
    for k in range(ROWS_PER_TILE // STAGE_ROWS):
        pltpu.sync_copy(stage_v, acc_sh.at[pl.ds(rbase + k * STAGE_ROWS, STAGE_ROWS)])
    plsc.subcore_barrier()

    def step(i, carry):
        base = ebase + i * CHUNK
        pltpu.sync_copy(src_hbm.at[pl.ds(base, CHUNK)], src_v)
        pltpu.sync_copy(dst_hbm.at[pl.ds(base, CHUNK)], dst_v)
        pltpu.async_copy(xa_hbm.at[src_v], rows_v, sem).wait()
        pltpu.sync_copy(rows_v, acc_sh.at[dst_v], add=True)
        return carry

    lax.fori_loop(0, N_CHUNKS, step, 0)
    plsc.subcore_barrier()

    for k in range(ROWS_PER_TILE // STAGE_ROWS):
        r0 = rbase + k * STAGE_ROWS
        pltpu.sync_copy(acc_sh.at[pl.ds(r0, STAGE_ROWS)], stage_v)
        pltpu.sync_copy(stage_v, parts_hbm.at[c, pl.ds(r0, STAGE_ROWS)])


_sc_aggregate = functools.partial(
    pl.kernel,
    out_type=jax.ShapeDtypeStruct((NUM_CORES, N_NODES, D_PAD), jnp.float32),
    mesh=plsc.VectorSubcoreMesh(core_axis_name="c", subcore_axis_name="s"),
    scratch_types=[
        pltpu.VMEM((CHUNK,), jnp.int32),
        pltpu.VMEM((CHUNK,), jnp.int32),
        pltpu.VMEM((CHUNK, D_PAD), jnp.float32),
        pltpu.VMEM((STAGE_ROWS, D_PAD), jnp.float32),
        pltpu.VMEM_SHARED((N_NODES, D_PAD), jnp.float32),
        pltpu.SemaphoreType.DMA,
    ],
)(_sc_body)


BLK = 400


def _tc_body(p_ref, x_ref, wl_ref, wr_ref, b_ref, o_ref):
    p = p_ref[0] + p_ref[1]
    summed = p[:, :D_IN]
    cnt = jnp.maximum(p[:, D_IN:D_IN + 1], 1.0)
    mean = summed / cnt
    o_ref[...] = (
        jnp.dot(mean, wl_ref[...], preferred_element_type=jnp.float32)
        + jnp.dot(x_ref[...], wr_ref[...], preferred_element_type=jnp.float32)
        + b_ref[...]
    )


def _tc_finish(parts, x, W_l, W_r, b2):
    return pl.pallas_call(
        _tc_body,
        grid=(N_NODES // BLK,),
        in_specs=[
            pl.BlockSpec((NUM_CORES, BLK, D_PAD), lambda i: (0, i, 0)),
            pl.BlockSpec((BLK, D_IN), lambda i: (i, 0)),
            pl.BlockSpec((D_IN, D_OUT), lambda i: (0, 0)),
            pl.BlockSpec((D_IN, D_OUT), lambda i: (0, 0)),
            pl.BlockSpec((1, D_OUT), lambda i: (0, 0)),
        ],
        out_specs=pl.BlockSpec((BLK, D_OUT), lambda i: (i, 0)),
        out_shape=jax.ShapeDtypeStruct((N_NODES, D_OUT), jnp.float32),
    )(parts, x, W_l, W_r, b2)


def kernel(x, edge_index, W_l, W_r, b_l):
    src = edge_index[0].astype(jnp.int32)
    dst = edge_index[1].astype(jnp.int32)
    xa = jnp.concatenate(
        [x,
         jnp.ones((N_NODES, 1), jnp.float32),
         jnp.zeros((N_NODES, D_PAD - D_IN - 1), jnp.float32)],
        axis=1,
    )
    zeros_blk = jnp.zeros((STAGE_ROWS, D_PAD), jnp.float32)
    parts = _sc_aggregate(xa, src, dst, zeros_blk)
    return _tc_finish(parts, x, W_l, W_r, jnp.reshape(b_l, (1, D_OUT)))


# speedup vs baseline: 5.5456x; 5.5456x over previous
import functools

import jax
import jax.numpy as jnp
from jax import lax
from jax.experimental import pallas as pl
from jax.experimental.pallas import tpu as pltpu
from jax.experimental.pallas import tpu_sc as plsc

N_NODES = 10000
N_EDGES = 320000
D_IN = 128
D_OUT = 128
D_PAD = 144

NUM_CORES = 2
NUM_SUBCORES = 16
NUM_TILES = NUM_CORES * NUM_SUBCORES

E_PER_TILE = N_EDGES // NUM_TILES
CHUNK = 80
N_CHUNKS = E_PER_TILE // CHUNK
ACC_ROWS = 10240
ROWS_PER_TILE = ACC_ROWS // NUM_SUBCORES
STAGE_ROWS = 128


def _sc_body(xa_hbm, src_hbm, dst_hbm, zeros_hbm, parts_hbm,
             src_v, dst_v, rows_v, stage_v, acc_sh, sem):
    c = lax.axis_index("c")
    s = lax.axis_index("s")
    wid = c * NUM_SUBCORES + s
    ebase = wid * E_PER_TILE
    rbase = s * ROWS_PER_TILE

    pltpu.sync_copy(zeros_hbm, stage_v)
    for k in range(ROWS_PER_TILE // STAGE_ROWS):
        r0 = pl.multiple_of(rbase + k * STAGE_ROWS, 8)
        pltpu.sync_copy(stage_v, acc_sh.at[pl.ds(r0, STAGE_ROWS)])
    plsc.subcore_barrier()

    def step(i, carry):
        base = pl.multiple_of(ebase + i * CHUNK, 8)
        pltpu.sync_copy(src_hbm.at[pl.ds(base, CHUNK)], src_v)
        pltpu.sync_copy(dst_hbm.at[pl.ds(base, CHUNK)], dst_v)
        pltpu.async_copy(xa_hbm.at[src_v], rows_v, sem).wait()
        pltpu.sync_copy(rows_v, acc_sh.at[dst_v], add=True)
        return carry

    lax.fori_loop(0, N_CHUNKS, step, 0)
    plsc.subcore_barrier()

    for k in range(ROWS_PER_TILE // STAGE_ROWS):
        r0 = pl.multiple_of(rbase + k * STAGE_ROWS, 8)
        pltpu.sync_copy(acc_sh.at[pl.ds(r0, STAGE_ROWS)], stage_v)
        pltpu.sync_copy(stage_v, parts_hbm.at[c, pl.ds(r0, STAGE_ROWS)])


_sc_aggregate = functools.partial(
    pl.kernel,
    out_type=jax.ShapeDtypeStruct((NUM_CORES, ACC_ROWS, D_PAD), jnp.float32),
    mesh=plsc.VectorSubcoreMesh(core_axis_name="c", subcore_axis_name="s"),
    compiler_params=pltpu.CompilerParams(use_tc_tiling_on_sc=False),
    scratch_types=[
        pltpu.VMEM((CHUNK,), jnp.int32),
        pltpu.VMEM((CHUNK,), jnp.int32),
        pltpu.VMEM((CHUNK, D_PAD), jnp.float32),
        pltpu.VMEM((STAGE_ROWS, D_PAD), jnp.float32),
        pltpu.VMEM_SHARED((ACC_ROWS, D_PAD), jnp.float32),
        pltpu.SemaphoreType.DMA,
    ],
)(_sc_body)


BLK = 400


def _tc_body(p_ref, x_ref, wl_ref, wr_ref, b_ref, o_ref):
    p = p_ref[0] + p_ref[1]
    summed = p[:, :D_IN]
    cnt = jnp.maximum(p[:, D_IN:D_IN + 1], 1.0)
    mean = summed / cnt
    o_ref[...] = (
        jnp.dot(mean, wl_ref[...], preferred_element_type=jnp.float32)
        + jnp.dot(x_ref[...], wr_ref[...], preferred_element_type=jnp.float32)
        + b_ref[...]
    )


def _tc_finish(parts, x, W_l, W_r, b2):
    return pl.pallas_call(
        _tc_body,
        grid=(N_NODES // BLK,),
        in_specs=[
            pl.BlockSpec((NUM_CORES, BLK, D_PAD), lambda i: (0, i, 0)),
            pl.BlockSpec((BLK, D_IN), lambda i: (i, 0)),
            pl.BlockSpec((D_IN, D_OUT), lambda i: (0, 0)),
            pl.BlockSpec((D_IN, D_OUT), lambda i: (0, 0)),
            pl.BlockSpec((1, D_OUT), lambda i: (0, 0)),
        ],
        out_specs=pl.BlockSpec((BLK, D_OUT), lambda i: (i, 0)),
        out_shape=jax.ShapeDtypeStruct((N_NODES, D_OUT), jnp.float32),
    )(parts, x, W_l, W_r, b2)


def kernel(x, edge_index, W_l, W_r, b_l):
    src = edge_index[0].astype(jnp.int32)
    dst = edge_index[1].astype(jnp.int32)
    xa = jnp.concatenate(
        [x,
         jnp.ones((N_NODES, 1), jnp.float32),
         jnp.zeros((N_NODES, D_PAD - D_IN - 1), jnp.float32)],
        axis=1,
    )
    zeros_blk = jnp.zeros((STAGE_ROWS, D_PAD), jnp.float32)
    parts = _sc_aggregate(xa, src, dst, zeros_blk)
    return _tc_finish(parts, x, W_l, W_r, jnp.reshape(b_l, (1, D_OUT)))


# speedup vs baseline: 9.7288x; 1.7543x over previous
import functools

import jax
import jax.numpy as jnp
from jax import lax
from jax.experimental import pallas as pl
from jax.experimental.pallas import tpu as pltpu
from jax.experimental.pallas import tpu_sc as plsc

N_NODES = 10000
N_EDGES = 320000
D_IN = 128
D_OUT = 128
D_PAD = 144

NUM_CORES = 2
NUM_SUBCORES = 16
NUM_TILES = NUM_CORES * NUM_SUBCORES

E_PER_TILE = N_EDGES // NUM_TILES
CHUNK = 125
N_CHUNKS = E_PER_TILE // CHUNK
ROWS_PER_TILE = N_NODES // NUM_SUBCORES
STAGE_K = ROWS_PER_TILE // CHUNK


def _sc_body(xa_hbm, src_hbm, dst_hbm, zeros_hbm, parts_hbm,
             srcb0, srcb1, dstb0, dstb1, rows0, rows1,
             acc_sh, isem0, isem1, gsem0, gsem1):
    c = lax.axis_index("c")
    s = lax.axis_index("s")
    wid = c * NUM_SUBCORES + s
    cbase = wid * N_CHUNKS
    rbase = s * ROWS_PER_TILE
    srcb = (srcb0, srcb1)
    dstb = (dstb0, dstb1)
    rows = (rows0, rows1)
    isems = (isem0, isem1)
    gsems = (gsem0, gsem1)

    pltpu.sync_copy(zeros_hbm, rows0)
    for k in range(STAGE_K):
        pltpu.sync_copy(rows0, acc_sh.at[pl.ds(rbase + k * CHUNK, CHUNK)])
    plsc.subcore_barrier()

    def start_idx(b, i):
        pltpu.async_copy(src_hbm.at[cbase + i], srcb[b], isems[b])
        pltpu.async_copy(dst_hbm.at[cbase + i], dstb[b], isems[b])

    def wait_idx(b, i):
        pltpu.make_async_copy(src_hbm.at[cbase + i], srcb[b], isems[b]).wait()
        pltpu.make_async_copy(dst_hbm.at[cbase + i], dstb[b], isems[b]).wait()

    def start_gather(b):
        pltpu.async_copy(xa_hbm.at[srcb[b]], rows[b], gsems[b])

    def wait_gather(b):
        pltpu.make_async_copy(xa_hbm.at[srcb[b]], rows[b], gsems[b]).wait()

    def scatter(b):
        pltpu.sync_copy(rows[b], acc_sh.at[dstb[b]], add=True)

    start_idx(0, 0)
    start_idx(1, 1)
    wait_idx(0, 0)
    start_gather(0)

    def step(j, carry):
        for b in range(2):
            i = 2 * j + b
            wait_idx(1 - b, i + 1)
            start_gather(1 - b)
            wait_gather(b)
            scatter(b)
            start_idx(b, i + 2)
        return carry

    lax.fori_loop(0, N_CHUNKS // 2 - 1, step, 0)
    wait_idx(1, N_CHUNKS - 1)
    start_gather(1)
    wait_gather(0)
    scatter(0)
    wait_gather(1)
    scatter(1)
    plsc.subcore_barrier()

    for k in range(STAGE_K):
        r0 = rbase + k * CHUNK
        pltpu.sync_copy(acc_sh.at[pl.ds(r0, CHUNK)], rows0)
        pltpu.sync_copy(rows0, parts_hbm.at[c, pl.ds(r0, CHUNK)])


_sc_aggregate = functools.partial(
    pl.kernel,
    out_type=jax.ShapeDtypeStruct((NUM_CORES, N_NODES, D_PAD), jnp.float32),
    mesh=plsc.VectorSubcoreMesh(core_axis_name="c", subcore_axis_name="s"),
    compiler_params=pltpu.CompilerParams(use_tc_tiling_on_sc=False),
    scratch_types=[
        pltpu.VMEM((CHUNK,), jnp.int32),
        pltpu.VMEM((CHUNK,), jnp.int32),
        pltpu.VMEM((CHUNK,), jnp.int32),
        pltpu.VMEM((CHUNK,), jnp.int32),
        pltpu.VMEM((CHUNK, D_PAD), jnp.float32),
        pltpu.VMEM((CHUNK, D_PAD), jnp.float32),
        pltpu.VMEM_SHARED((N_NODES, D_PAD), jnp.float32),
        pltpu.SemaphoreType.DMA,
        pltpu.SemaphoreType.DMA,
        pltpu.SemaphoreType.DMA,
        pltpu.SemaphoreType.DMA,
    ],
)(_sc_body)


BLK = 400


def _tc_body(p_ref, x_ref, wl_ref, wr_ref, b_ref, o_ref):
    p = p_ref[0] + p_ref[1]
    summed = p[:, :D_IN]
    cnt = jnp.maximum(p[:, D_IN:D_IN + 1], 1.0)
    mean = summed / cnt
    o_ref[...] = (
        jnp.dot(mean, wl_ref[...], preferred_element_type=jnp.float32)
        + jnp.dot(x_ref[...], wr_ref[...], preferred_element_type=jnp.float32)
        + b_ref[...]
    )


def _tc_finish(parts, x, W_l, W_r, b2):
    return pl.pallas_call(
        _tc_body,
        grid=(N_NODES // BLK,),
        in_specs=[
            pl.BlockSpec((NUM_CORES, BLK, D_PAD), lambda i: (0, i, 0)),
            pl.BlockSpec((BLK, D_IN), lambda i: (i, 0)),
            pl.BlockSpec((D_IN, D_OUT), lambda i: (0, 0)),
            pl.BlockSpec((D_IN, D_OUT), lambda i: (0, 0)),
            pl.BlockSpec((1, D_OUT), lambda i: (0, 0)),
        ],
        out_specs=pl.BlockSpec((BLK, D_OUT), lambda i: (i, 0)),
        out_shape=jax.ShapeDtypeStruct((N_NODES, D_OUT), jnp.float32),
    )(parts, x, W_l, W_r, b2)


def kernel(x, edge_index, W_l, W_r, b_l):
    src = edge_index[0].astype(jnp.int32).reshape(N_EDGES // CHUNK, CHUNK)
    dst = edge_index[1].astype(jnp.int32).reshape(N_EDGES // CHUNK, CHUNK)
    xa = jnp.concatenate(
        [x,
         jnp.ones((N_NODES, 1), jnp.float32),
         jnp.zeros((N_NODES, D_PAD - D_IN - 1), jnp.float32)],
        axis=1,
    )
    zeros_blk = jnp.zeros((CHUNK, D_PAD), jnp.float32)
    parts = _sc_aggregate(xa, src, dst, zeros_blk)
    return _tc_finish(parts, x, W_l, W_r, jnp.reshape(b_l, (1, D_OUT)))


# speedup vs baseline: 14.5275x; 1.4932x over previous
import functools

import jax
import jax.numpy as jnp
from jax import lax
from jax.experimental import pallas as pl
from jax.experimental.pallas import tpu as pltpu
from jax.experimental.pallas import tpu_sc as plsc

N_NODES = 10000
N_EDGES = 320000
D_IN = 128
D_OUT = 128

NUM_CORES = 2
NUM_SUBCORES = 16
NUM_TILES = NUM_CORES * NUM_SUBCORES

E_PER_TILE = N_EDGES // NUM_TILES
CHUNK = 80
N_CHUNKS = E_PER_TILE // CHUNK
ROWS_PER_TILE = N_NODES // NUM_SUBCORES
CNT_ROWS = 10240
CNT_STRIPE = CNT_ROWS // NUM_SUBCORES
NB_R = 3
NB_I = 4
UNROLL = 12
LOOP_CHUNKS = N_CHUNKS - 5


def _sc_body(x_hbm, ei_hbm, zeros2d_hbm, zeros1d_hbm, ones_hbm,
             parts_hbm, cnts_hbm,
             srcb0, srcb1, srcb2, srcb3, dstb0, dstb1, dstb2, dstb3,
             rows0, rows1, rows2, ones_v, cbuf, acc_sh, cnt_sh,
             isem0, isem1, isem2, isem3, gsem0, gsem1, gsem2,
             fsem0, fsem1, fsem2, csem0, csem1, csem2, csem3):
    c = lax.axis_index("c")
    s = lax.axis_index("s")
    wid = c * NUM_SUBCORES + s
    ebase = wid * E_PER_TILE
    abase = s * ROWS_PER_TILE
    cntbase = s * CNT_STRIPE
    srcb = (srcb0, srcb1, srcb2, srcb3)
    dstb = (dstb0, dstb1, dstb2, dstb3)
    rows = (rows0, rows1, rows2)
    isem = (isem0, isem1, isem2, isem3)
    gsem = (gsem0, gsem1, gsem2)
    fsem = (fsem0, fsem1, fsem2)
    csem = (csem0, csem1, csem2, csem3)

    pltpu.sync_copy(zeros2d_hbm, rows0)
    for k in range(7):
        pltpu.sync_copy(rows0, acc_sh.at[pl.ds(abase + k * CHUNK, CHUNK)])
    pltpu.sync_copy(rows0.at[pl.ds(0, 65)],
                    acc_sh.at[pl.ds(abase + 560, 65)])
    pltpu.sync_copy(zeros1d_hbm, cbuf)
    pltpu.sync_copy(cbuf, cnt_sh.at[pl.ds(cntbase, CNT_STRIPE)])
    pltpu.sync_copy(ones_hbm, ones_v)
    plsc.subcore_barrier()

    def idx_start(q, i):
        off = ebase + i * CHUNK
        pltpu.async_copy(ei_hbm.at[0, pl.ds(off, CHUNK)], srcb[q], isem[q])
        pltpu.async_copy(ei_hbm.at[1, pl.ds(off, CHUNK)], dstb[q], isem[q])

    def idx_wait(q):
        pltpu.make_async_copy(ei_hbm.at[0, pl.ds(0, CHUNK)], srcb[q], isem[q]).wait()
        pltpu.make_async_copy(ei_hbm.at[1, pl.ds(0, CHUNK)], dstb[q], isem[q]).wait()

    def g_start(r, q):
        pltpu.async_copy(x_hbm.at[srcb[q]], rows[r], gsem[r])

    def g_wait(r, q):
        pltpu.make_async_copy(x_hbm.at[srcb[q]], rows[r], gsem[r]).wait()

    def f_start(r, q):
        pltpu.async_copy(rows[r], acc_sh.at[dstb[q]], fsem[r], add=True)

    def f_wait(r, q):
        pltpu.make_async_copy(rows[r], acc_sh.at[dstb[q]], fsem[r]).wait()

    def c_start(q):
        pltpu.async_copy(ones_v, cnt_sh.at[dstb[q]], csem[q], add=True)

    def c_wait(q):
        pltpu.make_async_copy(ones_v, cnt_sh.at[dstb[q]], csem[q]).wait()

    def slot(i, roff, qoff):
        r, q = roff % NB_R, qoff % NB_I
        g_wait(r, q)
        f_start(r, q)
        c_start(q)
        if isinstance(i, int) and i == 0:
            pass
        else:
            f_wait((roff - 1) % NB_R, (qoff - 1) % NB_I)
            c_wait((qoff - 1) % NB_I)
        idx_wait((qoff + 2) % NB_I)
        g_start((roff + 2) % NB_R, (qoff + 2) % NB_I)
        nxt = jnp.minimum(i + 3, N_CHUNKS - 1)
        idx_start((qoff + 3) % NB_I, nxt)

    for q in range(3):
        idx_start(q, q)
    idx_wait(0)
    g_start(0, 0)
    idx_wait(1)
    g_start(1, 1)
    slot(0, 0, 0)
    slot(1, 1, 1)

    def step(j, carry):
        base = 2 + UNROLL * j
        for roff in range(UNROLL):
            slot(base + roff, 2 + roff, 2 + roff)
        return carry

    lax.fori_loop(0, LOOP_CHUNKS // UNROLL, step, 0)

    idx_wait((124) % NB_I)
    g_start((124) % NB_R, (124) % NB_I)
    for i in (122, 123, 124):
        r, q = i % NB_R, i % NB_I
        g_wait(r, q)
        f_start(r, q)
        c_start(q)
        f_wait((i - 1) % NB_R, (i - 1) % NB_I)
        c_wait((i - 1) % NB_I)
    f_wait(124 % NB_R, 124 % NB_I)
    c_wait(124 % NB_I)
    plsc.subcore_barrier()

    for k in range(7):
        r0 = abase + k * CHUNK
        pltpu.sync_copy(acc_sh.at[pl.ds(r0, CHUNK)], rows0)
        pltpu.sync_copy(rows0, parts_hbm.at[c, pl.ds(r0, CHUNK)])
    pltpu.sync_copy(acc_sh.at[pl.ds(abase + 560, 65)], rows0.at[pl.ds(0, 65)])
    pltpu.sync_copy(rows0.at[pl.ds(0, 65)],
                    parts_hbm.at[c, pl.ds(abase + 560, 65)])
    pltpu.sync_copy(cnt_sh.at[pl.ds(cntbase, CNT_STRIPE)], cbuf)
    pltpu.sync_copy(cbuf, cnts_hbm.at[c, pl.ds(cntbase, CNT_STRIPE)])


_sc_aggregate = functools.partial(
    pl.kernel,
    out_type=(
        jax.ShapeDtypeStruct((NUM_CORES, N_NODES, D_IN), jnp.float32),
        jax.ShapeDtypeStruct((NUM_CORES, CNT_ROWS), jnp.float32),
    ),
    mesh=plsc.VectorSubcoreMesh(core_axis_name="c", subcore_axis_name="s"),
    compiler_params=pltpu.CompilerParams(use_tc_tiling_on_sc=False),
    scratch_types=[
        pltpu.VMEM((CHUNK,), jnp.int32),
        pltpu.VMEM((CHUNK,), jnp.int32),
        pltpu.VMEM((CHUNK,), jnp.int32),
        pltpu.VMEM((CHUNK,), jnp.int32),
        pltpu.VMEM((CHUNK,), jnp.int32),
        pltpu.VMEM((CHUNK,), jnp.int32),
        pltpu.VMEM((CHUNK,), jnp.int32),
        pltpu.VMEM((CHUNK,), jnp.int32),
        pltpu.VMEM((CHUNK, D_IN), jnp.float32),
        pltpu.VMEM((CHUNK, D_IN), jnp.float32),
        pltpu.VMEM((CHUNK, D_IN), jnp.float32),
        pltpu.VMEM((CHUNK,), jnp.float32),
        pltpu.VMEM((CNT_STRIPE,), jnp.float32),
        pltpu.VMEM_SHARED((N_NODES, D_IN), jnp.float32),
        pltpu.VMEM_SHARED((CNT_ROWS,), jnp.float32),
    ] + [pltpu.SemaphoreType.DMA] * 14,
)(_sc_body)


BLK = 400


def _tc_body(p_ref, cn_ref, x_ref, wl_ref, wr_ref, b_ref, o_ref):
    summed = p_ref[0] + p_ref[1]
    cnt = jnp.maximum(cn_ref[0] + cn_ref[1], 1.0)
    mean = summed / cnt
    o_ref[...] = (
        jnp.dot(mean, wl_ref[...], preferred_element_type=jnp.float32)
        + jnp.dot(x_ref[...], wr_ref[...], preferred_element_type=jnp.float32)
        + b_ref[...]
    )


def _tc_finish(parts, cnts, x, W_l, W_r, b2):
    return pl.pallas_call(
        _tc_body,
        grid=(N_NODES // BLK,),
        in_specs=[
            pl.BlockSpec((NUM_CORES, BLK, D_IN), lambda i: (0, i, 0)),
            pl.BlockSpec((NUM_CORES, BLK, 1), lambda i: (0, i, 0)),
            pl.BlockSpec((BLK, D_IN), lambda i: (i, 0)),
            pl.BlockSpec((D_IN, D_OUT), lambda i: (0, 0)),
            pl.BlockSpec((D_IN, D_OUT), lambda i: (0, 0)),
            pl.BlockSpec((1, D_OUT), lambda i: (0, 0)),
        ],
        out_specs=pl.BlockSpec((BLK, D_OUT), lambda i: (i, 0)),
        out_shape=jax.ShapeDtypeStruct((N_NODES, D_OUT), jnp.float32),
    )(parts, cnts, x, W_l, W_r, b2)


def kernel(x, edge_index, W_l, W_r, b_l):
    ei = edge_index.astype(jnp.int32)
    zeros2d = jnp.zeros((CHUNK, D_IN), jnp.float32)
    zeros1d = jnp.zeros((CNT_STRIPE,), jnp.float32)
    ones1 = jnp.ones((CHUNK,), jnp.float32)
    parts, cnts = _sc_aggregate(x, ei, zeros2d, zeros1d, ones1)
    cnts3 = cnts.reshape(NUM_CORES, CNT_ROWS, 1)
    return _tc_finish(parts, cnts3, x, W_l, W_r, jnp.reshape(b_l, (1, D_OUT)))
